# baseline (speedup 1.0000x reference)
import functools
import math

import numpy as np

import jax
import jax.numpy as jnp
from jax import lax
from jax.experimental import pallas as pl
from jax.experimental.pallas import tpu as pltpu
from jax.experimental.pallas import tpu_sc as plsc

_LANES = 16


def _sinusoidal_pe_np(max_len, d):
    pos = np.arange(max_len, dtype=np.float32)[:, None]
    i = np.arange(0, d, 2, dtype=np.float32)
    div = np.exp((-math.log(10000.0) * i / d).astype(np.float32))
    ang = pos * div[None, :]
    pe = np.zeros((max_len, d), dtype=np.float32)
    pe[:, 0::2] = np.sin(ang)
    pe[:, 1::2] = np.cos(ang)
    return pe


def _build_sc_kernel(b, s, v, d, num_cores, num_subcores):
    nw = num_cores * num_subcores
    n = b * s
    per_w = n // nw
    ch = 64
    n_ch = per_w // ch
    spans_per_b = s // per_w
    scale = float(np.float32(math.sqrt(d)))
    mesh = plsc.VectorSubcoreMesh(core_axis_name="c", subcore_axis_name="s")

    @functools.partial(
        pl.kernel,
        mesh=mesh,
        out_type=jax.ShapeDtypeStruct((b, s, d), jnp.float32),
        scratch_types=[
            pltpu.VMEM((per_w,), jnp.int32),
            pltpu.VMEM((per_w, d), jnp.float32),
            pltpu.VMEM((per_w, d), jnp.float32),
            pltpu.SemaphoreType.DMA,
            pltpu.SemaphoreType.DMA,
            pltpu.SemaphoreType.DMA,
            pltpu.SemaphoreType.DMA,
            pltpu.SemaphoreType.DMA,
            pltpu.SemaphoreType.DMA,
        ],
    )
    def emb_kernel(idx_hbm, table_hbm, pe_hbm, out_hbm,
                   idx_v, rows_v, pe_v,
                   g0, g1, g2, g3, psem, wsem):
        wid = lax.axis_index("s") * num_cores + lax.axis_index("c")
        bi = wid // spans_per_b
        s_base = lax.rem(wid, spans_per_b) * per_w

        pe_cps = [pltpu.async_copy(pe_hbm.at[pl.ds(s_base + c * ch, ch)],
                                   pe_v.at[pl.ds(c * ch, ch)], psem)
                  for c in range(n_ch)]
        pltpu.sync_copy(idx_hbm.at[bi, pl.ds(s_base, per_w)], idx_v)
        gsems = [g0, g1, g2, g3]
        gathers = [
            pltpu.async_copy(table_hbm.at[idx_v.at[pl.ds(c * ch, ch)]],
                             rows_v.at[pl.ds(c * ch, ch)], gsems[c])
            for c in range(n_ch)
        ]

        writes = []
        for c in range(n_ch):
            pe_cps[c].wait()
            gathers[c].wait()

            @plsc.parallel_loop(0, ch, unroll=2)
            def _fma(i, _c=c):
                row = _c * ch + i
                for j in range(d // _LANES):
                    sl = pl.ds(j * _LANES, _LANES)
                    plsc.addupdate(pe_v.at[row, sl], rows_v[row, sl] * scale)

            writes.append(pltpu.async_copy(
                pe_v.at[pl.ds(c * ch, ch)],
                out_hbm.at[bi, pl.ds(s_base + c * ch, ch)], wsem))
        for w in writes:
            w.wait()

    return emb_kernel


def kernel(input_ids, token_table):
    b, s = input_ids.shape
    v, d = token_table.shape
    info = plsc.get_sparse_core_info()
    zero = lax.optimization_barrier(jnp.float32(0.0))
    pe = jnp.asarray(_sinusoidal_pe_np(s, d)) + zero
    emb = _build_sc_kernel(b, s, v, d, info.num_cores, info.num_subcores)
    return emb(input_ids, token_table, pe)


# speedup vs baseline: 1.0367x; 1.0367x over previous
import functools
import math

import numpy as np

import jax
import jax.numpy as jnp
from jax import lax
from jax.experimental import pallas as pl
from jax.experimental.pallas import tpu as pltpu
from jax.experimental.pallas import tpu_sc as plsc

_LANES = 16


def _sinusoidal_pe_np(max_len, d):
    pos = np.arange(max_len, dtype=np.float32)[:, None]
    i = np.arange(0, d, 2, dtype=np.float32)
    div = np.exp((-math.log(10000.0) * i / d).astype(np.float32))
    ang = pos * div[None, :]
    pe = np.zeros((max_len, d), dtype=np.float32)
    pe[:, 0::2] = np.sin(ang)
    pe[:, 1::2] = np.cos(ang)
    return pe


def _pack_pe_bf16_words(pe):
    s, d = pe.shape
    g = pe.reshape(s, d // 32, 2, 16)
    lo, hi = g[:, :, 0, :], g[:, :, 1, :]

    def to_bf16_bits(x):
        u = x.astype(np.float32).view(np.uint32)
        return ((u + 0x7FFF + ((u >> 16) & 1)) >> 16).astype(np.uint32)

    words = to_bf16_bits(lo) | (to_bf16_bits(hi) << np.uint32(16))
    return words.reshape(s * d // 2).view(np.int32)


def _build_sc_kernel(b, s, v, d, num_cores, num_subcores):
    nw = num_cores * num_subcores
    n = b * s
    per_w = n // nw
    ch = 64
    n_ch = per_w // ch
    spans_per_b = s // per_w
    scale = float(np.float32(math.sqrt(d)))
    mesh = plsc.VectorSubcoreMesh(core_axis_name="c", subcore_axis_name="s")

    @functools.partial(
        pl.kernel,
        mesh=mesh,
        out_type=jax.ShapeDtypeStruct((b, s, d), jnp.float32),
        scratch_types=[
            pltpu.VMEM((per_w,), jnp.int32),
            pltpu.VMEM((per_w, d), jnp.float32),
            pltpu.VMEM((per_w * d // 2,), jnp.int32),
            pltpu.SemaphoreType.DMA,
            pltpu.SemaphoreType.DMA,
            pltpu.SemaphoreType.DMA,
            pltpu.SemaphoreType.DMA,
            pltpu.SemaphoreType.DMA,
            pltpu.SemaphoreType.DMA,
        ],
    )
    def emb_kernel(idx_hbm, table_hbm, pe_hbm, out_hbm,
                   idx_v, rows_v, pe_v,
                   g0, g1, g2, g3, psem, wsem):
        wid = lax.axis_index("s") * num_cores + lax.axis_index("c")
        bi = wid // spans_per_b
        s_base = lax.rem(wid, spans_per_b) * per_w

        pe_base = pl.multiple_of(s_base * (d // 2), 8)
        pe_cps = [pltpu.async_copy(
            pe_hbm.at[pl.ds(pe_base + c * ch * d // 2, ch * d // 2)],
            pe_v.at[pl.ds(c * ch * d // 2, ch * d // 2)], psem)
                  for c in range(n_ch)]
        pltpu.sync_copy(idx_hbm.at[bi, pl.ds(s_base, per_w)], idx_v)
        gsems = [g0, g1, g2, g3]
        gathers = [
            pltpu.async_copy(table_hbm.at[idx_v.at[pl.ds(c * ch, ch)]],
                             rows_v.at[pl.ds(c * ch, ch)], gsems[c])
            for c in range(n_ch)
        ]

        writes = []
        for c in range(n_ch):
            pe_cps[c].wait()
            gathers[c].wait()

            @plsc.parallel_loop(0, ch, unroll=2)
            def _fma(i, _c=c):
                row = _c * ch + i
                for g in range(d // (2 * _LANES)):
                    woff = pl.multiple_of(row * (d // 2), 8)
                    w = pe_v[pl.ds(woff + g * _LANES, _LANES)]
                    pa = lax.bitcast_convert_type(w << 16, jnp.float32)
                    pb = lax.bitcast_convert_type(w & jnp.int32(-65536),
                                                  jnp.float32)
                    sla = pl.ds(g * 2 * _LANES, _LANES)
                    slb = pl.ds(g * 2 * _LANES + _LANES, _LANES)
                    rows_v[row, sla] = rows_v[row, sla] * scale + pa
                    rows_v[row, slb] = rows_v[row, slb] * scale + pb

            writes.append(pltpu.async_copy(
                rows_v.at[pl.ds(c * ch, ch)],
                out_hbm.at[bi, pl.ds(s_base + c * ch, ch)], wsem))
        for w in writes:
            w.wait()

    return emb_kernel


def kernel(input_ids, token_table):
    b, s = input_ids.shape
    v, d = token_table.shape
    info = plsc.get_sparse_core_info()
    pe_np = _pack_pe_bf16_words(_sinusoidal_pe_np(s, d))
    zero = lax.optimization_barrier(jnp.int32(0))
    pe = jnp.asarray(pe_np) | zero
    emb = _build_sc_kernel(b, s, v, d, info.num_cores, info.num_subcores)
    return emb(input_ids, token_table, pe)


# speedup vs baseline: 1.0462x; 1.0092x over previous
import functools
import math

import numpy as np

import jax
import jax.numpy as jnp
from jax import lax
from jax.experimental import pallas as pl
from jax.experimental.pallas import tpu as pltpu
from jax.experimental.pallas import tpu_sc as plsc

_LANES = 16


def _sinusoidal_pe_np(max_len, d):
    pos = np.arange(max_len, dtype=np.float32)[:, None]
    i = np.arange(0, d, 2, dtype=np.float32)
    div = np.exp((-math.log(10000.0) * i / d).astype(np.float32))
    ang = pos * div[None, :]
    pe = np.zeros((max_len, d), dtype=np.float32)
    pe[:, 0::2] = np.sin(ang)
    pe[:, 1::2] = np.cos(ang)
    return pe


def _pack_pe_bf16_words(pe):
    s, d = pe.shape
    g = pe.reshape(s, d // 32, 2, 16)
    lo, hi = g[:, :, 0, :], g[:, :, 1, :]

    def to_bf16_bits(x):
        u = x.astype(np.float32).view(np.uint32)
        return ((u + 0x7FFF + ((u >> 16) & 1)) >> 16).astype(np.uint32)

    words = to_bf16_bits(lo) | (to_bf16_bits(hi) << np.uint32(16))
    return words.reshape(s * d // 2).view(np.int32)


def _build_sc_kernel(b, s, v, d, num_cores, num_subcores):
    nw = num_cores * num_subcores
    n = b * s
    per_w = n // nw
    ch = 128
    n_ch = per_w // ch
    spans_per_b = s // per_w
    scale = float(np.float32(math.sqrt(d)))
    mesh = plsc.VectorSubcoreMesh(core_axis_name="c", subcore_axis_name="s")

    @functools.partial(
        pl.kernel,
        mesh=mesh,
        out_type=jax.ShapeDtypeStruct((b, s, d), jnp.float32),
        scratch_types=[
            pltpu.VMEM((per_w,), jnp.int32),
            pltpu.VMEM((per_w, d), jnp.float32),
            pltpu.VMEM((per_w * d // 2,), jnp.int32),
            pltpu.SemaphoreType.DMA,
            pltpu.SemaphoreType.DMA,
            pltpu.SemaphoreType.DMA,
            pltpu.SemaphoreType.DMA,
            pltpu.SemaphoreType.DMA,
            pltpu.SemaphoreType.DMA,
        ],
    )
    def emb_kernel(idx_hbm, table_hbm, pe_hbm, out_hbm,
                   idx_v, rows_v, pe_v,
                   g0, g1, g2, g3, psem, wsem):
        wid = lax.axis_index("s") * num_cores + lax.axis_index("c")
        bi = wid // spans_per_b
        s_base = lax.rem(wid, spans_per_b) * per_w

        pe_base = pl.multiple_of(s_base * (d // 2), 8)
        pe_cps = [pltpu.async_copy(
            pe_hbm.at[pl.ds(pe_base + c * ch * d // 2, ch * d // 2)],
            pe_v.at[pl.ds(c * ch * d // 2, ch * d // 2)], psem)
                  for c in range(n_ch)]
        pltpu.sync_copy(idx_hbm.at[bi, pl.ds(s_base, per_w)], idx_v)
        gsems = [g0, g1, g2, g3][:n_ch]
        gathers = [
            pltpu.async_copy(table_hbm.at[idx_v.at[pl.ds(c * ch, ch)]],
                             rows_v.at[pl.ds(c * ch, ch)], gsems[c])
            for c in range(n_ch)
        ]

        writes = []
        for c in range(n_ch):
            pe_cps[c].wait()
            gathers[c].wait()

            @plsc.parallel_loop(0, ch, unroll=2)
            def _fma(i, _c=c):
                row = _c * ch + i
                for g in range(d // (2 * _LANES)):
                    woff = pl.multiple_of(row * (d // 2), 8)
                    w = pe_v[pl.ds(woff + g * _LANES, _LANES)]
                    pa = lax.bitcast_convert_type(w << 16, jnp.float32)
                    pb = lax.bitcast_convert_type(w & jnp.int32(-65536),
                                                  jnp.float32)
                    sla = pl.ds(g * 2 * _LANES, _LANES)
                    slb = pl.ds(g * 2 * _LANES + _LANES, _LANES)
                    rows_v[row, sla] = rows_v[row, sla] * scale + pa
                    rows_v[row, slb] = rows_v[row, slb] * scale + pb

            writes.append(pltpu.async_copy(
                rows_v.at[pl.ds(c * ch, ch)],
                out_hbm.at[bi, pl.ds(s_base + c * ch, ch)], wsem))
        for w in writes:
            w.wait()

    return emb_kernel


def kernel(input_ids, token_table):
    b, s = input_ids.shape
    v, d = token_table.shape
    info = plsc.get_sparse_core_info()
    pe_np = _pack_pe_bf16_words(_sinusoidal_pe_np(s, d))
    zero = lax.optimization_barrier(jnp.int32(0))
    pe = jnp.asarray(pe_np) | zero
    emb = _build_sc_kernel(b, s, v, d, info.num_cores, info.num_subcores)
    return emb(input_ids, token_table, pe)


# speedup vs baseline: 1.0549x; 1.0083x over previous
import functools
import math

import numpy as np

import jax
import jax.numpy as jnp
from jax import lax
from jax.experimental import pallas as pl
from jax.experimental.pallas import tpu as pltpu
from jax.experimental.pallas import tpu_sc as plsc

_LANES = 16


def _sinusoidal_pe_np(max_len, d):
    pos = np.arange(max_len, dtype=np.float32)[:, None]
    i = np.arange(0, d, 2, dtype=np.float32)
    div = np.exp((-math.log(10000.0) * i / d).astype(np.float32))
    ang = pos * div[None, :]
    pe = np.zeros((max_len, d), dtype=np.float32)
    pe[:, 0::2] = np.sin(ang)
    pe[:, 1::2] = np.cos(ang)
    return pe


def _pack_pe_bf16_words(pe):
    s, d = pe.shape
    g = pe.reshape(s, d // 32, 2, 16)
    lo, hi = g[:, :, 0, :], g[:, :, 1, :]

    def to_bf16_bits(x):
        u = x.astype(np.float32).view(np.uint32)
        return ((u + 0x7FFF + ((u >> 16) & 1)) >> 16).astype(np.uint32)

    words = to_bf16_bits(lo) | (to_bf16_bits(hi) << np.uint32(16))
    return words.reshape(s * d // 2).view(np.int32)


def _build_sc_kernel(b, s, v, d, num_cores, num_subcores):
    nw = num_cores * num_subcores
    n = b * s
    per_w = n // nw
    ch = 128
    n_ch = per_w // ch
    spans_per_b = s // per_w
    scale = float(np.float32(math.sqrt(d)))
    mesh = plsc.VectorSubcoreMesh(core_axis_name="c", subcore_axis_name="s")

    @functools.partial(
        pl.kernel,
        mesh=mesh,
        out_type=jax.ShapeDtypeStruct((b, s, d), jnp.float32),
        scratch_types=[
            pltpu.VMEM((per_w,), jnp.int32),
            pltpu.VMEM((per_w, d), jnp.float32),
            pltpu.VMEM((per_w * d // 2,), jnp.int32),
            pltpu.SemaphoreType.DMA,
            pltpu.SemaphoreType.DMA,
            pltpu.SemaphoreType.DMA,
            pltpu.SemaphoreType.DMA,
            pltpu.SemaphoreType.DMA,
            pltpu.SemaphoreType.DMA,
        ],
    )
    def emb_kernel(idx_hbm, table_hbm, pe_hbm, out_hbm,
                   idx_v, rows_v, pe_v,
                   g0, g1, g2, g3, psem, wsem):
        wid = lax.axis_index("s") * num_cores + lax.axis_index("c")
        bi = wid // spans_per_b
        s_base = lax.rem(wid, spans_per_b) * per_w

        pe_base = pl.multiple_of(s_base * (d // 2), 8)
        pe_cps = [pltpu.async_copy(
            pe_hbm.at[pl.ds(pe_base + c * ch * d // 2, ch * d // 2)],
            pe_v.at[pl.ds(c * ch * d // 2, ch * d // 2)], psem)
                  for c in range(n_ch)]
        pltpu.sync_copy(idx_hbm.at[bi, pl.ds(s_base, per_w)], idx_v)
        gsems = [g0, g1, g2, g3][:n_ch]
        gathers = [
            pltpu.async_copy(table_hbm.at[idx_v.at[pl.ds(c * ch, ch)]],
                             rows_v.at[pl.ds(c * ch, ch)], gsems[c])
            for c in range(n_ch)
        ]

        writes = []
        for c in range(n_ch):
            pe_cps[c].wait()
            gathers[c].wait()

            @plsc.parallel_loop(0, ch, unroll=1)
            def _fma(i, _c=c):
                row = _c * ch + i
                for g in range(d // (2 * _LANES)):
                    woff = pl.multiple_of(row * (d // 2), 8)
                    w = pe_v[pl.ds(woff + g * _LANES, _LANES)]
                    pa = lax.bitcast_convert_type(w << 16, jnp.float32)
                    pb = lax.bitcast_convert_type(w & jnp.int32(-65536),
                                                  jnp.float32)
                    sla = pl.ds(g * 2 * _LANES, _LANES)
                    slb = pl.ds(g * 2 * _LANES + _LANES, _LANES)
                    rows_v[row, sla] = rows_v[row, sla] * scale + pa
                    rows_v[row, slb] = rows_v[row, slb] * scale + pb

            writes.append(pltpu.async_copy(
                rows_v.at[pl.ds(c * ch, ch)],
                out_hbm.at[bi, pl.ds(s_base + c * ch, ch)], wsem))
        for w in writes:
            w.wait()

    return emb_kernel


def kernel(input_ids, token_table):
    b, s = input_ids.shape
    v, d = token_table.shape
    info = plsc.get_sparse_core_info()
    pe_np = _pack_pe_bf16_words(_sinusoidal_pe_np(s, d))
    zero = lax.optimization_barrier(jnp.int32(0))
    pe = jnp.asarray(pe_np) | zero
    emb = _build_sc_kernel(b, s, v, d, info.num_cores, info.num_subcores)
    return emb(input_ids, token_table, pe)
